# baseline (speedup 1.0000x reference)
import jax
import jax.numpy as jnp
from jax import lax
from jax.experimental import pallas as pl
from jax.experimental.pallas import tpu as pltpu
from jax.experimental.pallas import tpu_sc as plsc

_ROWS = 4096 * 50
_NC, _NS = 2, 16
_NW = _NC * _NS
_RPW = _ROWS // _NW
_R = 800
_NCHUNK = _RPW // _R

_mesh = plsc.VectorSubcoreMesh(core_axis_name="c", subcore_axis_name="s")

_out_t = jax.ShapeDtypeStruct((_ROWS, 32), jnp.float32)


def _body(x_hbm, m_hbm, t_hbm, v_hbm, s_hbm, b0, b1, b2, b3, sem):
    outs = (m_hbm, t_hbm, v_hbm, s_hbm)
    bufs = (b0, b1, b2, b3)
    wid = lax.axis_index("s") * _NC + lax.axis_index("c")
    base = wid * _RPW

    def chunk(c, _):
        rb = base + c * _R
        cps = [
            pltpu.async_copy(x_hbm.at[pl.ds(rb, _R), k], bufs[k], sem)
            for k in range(4)
        ]
        for k in range(4):
            cps[k].wait()
            pltpu.sync_copy(bufs[k], outs[k].at[pl.ds(rb, _R)])
        return ()

    lax.fori_loop(0, _NCHUNK, chunk, (), unroll=False)


_split = pl.kernel(
    _body,
    out_type=(_out_t,) * 4,
    mesh=_mesh,
    scratch_types=[pltpu.VMEM((_R, 32), jnp.float32) for _ in range(4)]
    + [pltpu.SemaphoreType.DMA],
    compiler_params=pltpu.CompilerParams(use_tc_tiling_on_sc=False),
)


@jax.jit
def kernel(x):
    xr = x.reshape(_ROWS, 4, 32)
    m, t, v, s = _split(xr)
    shp = (4096, 50, 32)
    return (m.reshape(shp), t.reshape(shp), v.reshape(shp), s.reshape(shp))
